# baseline (speedup 1.0000x reference)
import jax
import jax.numpy as jnp
from jax import lax
from jax.experimental import pallas as pl
from jax.experimental.pallas import tpu as pltpu
from jax.experimental.pallas import tpu_sc as plsc

BATCH = 16384
K = 16
NROWS = 1_000_000

_info = plsc.get_sparse_core_info()
NC, NS, L = _info.num_cores, _info.num_subcores, _info.num_lanes
NW = NC * NS
BPW = BATCH // NW
NG = BPW // L


def _body(users_hbm, items_hbm, gu_flat, gi_flat,
          xui_hbm, gu_out_hbm, gi_out_hbm,
          idx_u, idx_i, rows_u, rows_i, xui_v, sem_u, sem_i, sem_o):
    wid = lax.axis_index("s") * NC + lax.axis_index("c")
    base = wid * BPW

    pltpu.sync_copy(users_hbm.at[pl.ds(base, BPW)], idx_u)
    pltpu.sync_copy(items_hbm.at[pl.ds(base, BPW)], idx_i)

    def gather(g, carry):
        s = g * L
        au = idx_u[pl.ds(s, L)]
        ai = idx_i[pl.ds(s, L)]
        for c in range(K):
            pltpu.async_copy(
                gu_flat.at[au], rows_u.at[pl.ds(c * BPW + s, L)], sem_u)
            pltpu.async_copy(
                gi_flat.at[ai], rows_i.at[pl.ds(c * BPW + s, L)], sem_i)
            if c + 1 < K:
                au = au + NROWS
                ai = ai + NROWS
        return carry

    lax.fori_loop(0, NG, gather, 0)

    pltpu.make_async_copy(
        gu_flat.at[pl.ds(0, K * BPW)], rows_u, sem_u).wait()
    pltpu.make_async_copy(
        gi_flat.at[pl.ds(0, K * BPW)], rows_i, sem_i).wait()

    def blk(g, carry):
        s = g * L
        acc = rows_u[pl.ds(s, L)] * rows_i[pl.ds(s, L)]
        for c in range(1, K):
            acc = acc + (rows_u[pl.ds(c * BPW + s, L)]
                         * rows_i[pl.ds(c * BPW + s, L)])
        xui_v[pl.ds(s, L)] = acc
        return carry

    lax.fori_loop(0, NG, blk, 0)

    copies = []
    for c in range(K):
        copies.append(pltpu.async_copy(
            rows_u.at[pl.ds(c * BPW, BPW)],
            gu_out_hbm.at[c, pl.ds(base, BPW)], sem_o))
        copies.append(pltpu.async_copy(
            rows_i.at[pl.ds(c * BPW, BPW)],
            gi_out_hbm.at[c, pl.ds(base, BPW)], sem_o))
    copies.append(pltpu.async_copy(xui_v, xui_hbm.at[pl.ds(base, BPW)], sem_o))
    for cp in copies:
        cp.wait()


@jax.jit
def _run(users, items, gu_flat, gi_flat):
    mesh = plsc.VectorSubcoreMesh(core_axis_name="c", subcore_axis_name="s")
    f = pl.kernel(
        _body,
        mesh=mesh,
        compiler_params=pltpu.CompilerParams(use_tc_tiling_on_sc=False),
        out_type=(
            jax.ShapeDtypeStruct((BATCH,), jnp.float32),
            jax.ShapeDtypeStruct((K, BATCH), jnp.float32),
            jax.ShapeDtypeStruct((K, BATCH), jnp.float32),
        ),
        scratch_types=[
            pltpu.VMEM((BPW,), jnp.int32),
            pltpu.VMEM((BPW,), jnp.int32),
            pltpu.VMEM((K * BPW,), jnp.float32),
            pltpu.VMEM((K * BPW,), jnp.float32),
            pltpu.VMEM((BPW,), jnp.float32),
            pltpu.SemaphoreType.DMA,
            pltpu.SemaphoreType.DMA,
            pltpu.SemaphoreType.DMA,
        ],
    )
    return f(users, items, gu_flat, gi_flat)


def kernel(inputs, Gu, Gi):
    users = inputs[0]
    items = inputs[1]
    gu_flat = Gu.T.reshape(NROWS * K)
    gi_flat = Gi.T.reshape(NROWS * K)
    xui, gu_out_t, gi_out_t = _run(users, items, gu_flat, gi_flat)
    return xui, gu_out_t.T, gi_out_t.T


# speedup vs baseline: 3.0444x; 3.0444x over previous
import jax
import jax.numpy as jnp
from jax import lax
from jax.experimental import pallas as pl
from jax.experimental.pallas import tpu as pltpu
from jax.experimental.pallas import tpu_sc as plsc

BATCH = 16384
K = 16
NROWS = 1_000_000
GW = 128
NGRP = NROWS // 8

_info = plsc.get_sparse_core_info()
NC, NS, L = _info.num_cores, _info.num_subcores, _info.num_lanes
NW = NC * NS
BPW = BATCH // NW
WAVE = 256
NWAVE = BPW // WAVE


def _body(users_hbm, items_hbm, gu_g, gi_g,
          xui_hbm, gu_pad_hbm, gi_pad_hbm,
          idx_u, idx_i, idxg, raw, outpad, trans_u, trans_i, xui_v,
          sem_g, sem_o):
    wid = lax.axis_index("s") * NC + lax.axis_index("c")
    base = wid * BPW

    pltpu.sync_copy(users_hbm.at[pl.ds(base, BPW)], idx_u)
    pltpu.sync_copy(items_hbm.at[pl.ds(base, BPW)], idx_i)

    lanes = lax.iota(jnp.int32, L)

    def do_table(idx_ref, table_g, pad_hbm, trans):
        for w in range(NWAVE):
            woff = w * WAVE

            def grp(g, carry):
                s = g * L
                v = idx_ref[pl.ds(woff + s, L)]
                idxg[pl.ds(s, L)] = lax.shift_right_logical(v, 3)
                return carry

            lax.fori_loop(0, WAVE // L, grp, 0)

            g0 = pltpu.async_copy(
                table_g.at[idxg.at[pl.ds(0, 128)]],
                raw.at[pl.ds(0, 128), :], sem_g)
            g1 = pltpu.async_copy(
                table_g.at[idxg.at[pl.ds(128, 128)]],
                raw.at[pl.ds(128, 128), :], sem_g)
            g0.wait()
            g1.wait()

            def ext(g, carry):
                s = g * L
                jvec = s + lanes
                v = idx_ref[pl.ds(woff + s, L)]
                coff = lax.shift_left(
                    lax.bitwise_and(v, jnp.int32(7)), 4)
                for c in range(K):
                    vals = plsc.load_gather(raw, [jvec, coff + c])
                    trans[pl.ds(c * BPW + woff + s, L)] = vals
                    plsc.store_scatter(
                        outpad, [jvec, jnp.full((L,), c, jnp.int32)], vals)
                return carry

            lax.fori_loop(0, WAVE // L, ext, 0)

            pltpu.sync_copy(
                outpad, pad_hbm.at[pl.ds(base + woff, WAVE), :])

    do_table(idx_u, gu_g, gu_pad_hbm, trans_u)
    do_table(idx_i, gi_g, gi_pad_hbm, trans_i)

    def blk(g, carry):
        s = g * L
        acc = trans_u[pl.ds(s, L)] * trans_i[pl.ds(s, L)]
        for c in range(1, K):
            acc = acc + (trans_u[pl.ds(c * BPW + s, L)]
                         * trans_i[pl.ds(c * BPW + s, L)])
        xui_v[pl.ds(s, L)] = acc
        return carry

    lax.fori_loop(0, BPW // L, blk, 0)

    pltpu.sync_copy(xui_v, xui_hbm.at[pl.ds(base, BPW)])


@jax.jit
def _run(users, items, gu_g, gi_g):
    mesh = plsc.VectorSubcoreMesh(core_axis_name="c", subcore_axis_name="s")
    f = pl.kernel(
        _body,
        mesh=mesh,
        compiler_params=pltpu.CompilerParams(
            needs_layout_passes=False, use_tc_tiling_on_sc=True),
        out_type=(
            jax.ShapeDtypeStruct((BATCH,), jnp.float32),
            jax.ShapeDtypeStruct((BATCH, GW), jnp.float32),
            jax.ShapeDtypeStruct((BATCH, GW), jnp.float32),
        ),
        scratch_types=[
            pltpu.VMEM((BPW,), jnp.int32),
            pltpu.VMEM((BPW,), jnp.int32),
            pltpu.VMEM((WAVE,), jnp.int32),
            pltpu.VMEM((WAVE, GW), jnp.float32),
            pltpu.VMEM((WAVE, GW), jnp.float32),
            pltpu.VMEM((K * BPW,), jnp.float32),
            pltpu.VMEM((K * BPW,), jnp.float32),
            pltpu.VMEM((BPW,), jnp.float32),
            pltpu.SemaphoreType.DMA,
            pltpu.SemaphoreType.DMA,
        ],
    )
    return f(users, items, gu_g, gi_g)


def kernel(inputs, Gu, Gi):
    users = inputs[0]
    items = inputs[1]
    gu_g = Gu.reshape(NGRP, GW)
    gi_g = Gi.reshape(NGRP, GW)
    xui, gu_pad, gi_pad = _run(users, items, gu_g, gi_g)
    return xui, gu_pad[:, :K], gi_pad[:, :K]


# speedup vs baseline: 14.4643x; 4.7512x over previous
import jax
import jax.numpy as jnp
from jax import lax
from jax.experimental import pallas as pl
from jax.experimental.pallas import tpu as pltpu
from jax.experimental.pallas import tpu_sc as plsc

BATCH = 16384
K = 16
NROWS = 1_000_000
GW = 128

_info = plsc.get_sparse_core_info()
NC, NS, L = _info.num_cores, _info.num_subcores, _info.num_lanes
NW = NC * NS
BPW = BATCH // NW
WAVE = 256
RING = 16


def _body(users_hbm, items_hbm, gu_t, gi_t,
          xui_hbm, gu_pad_hbm, gi_pad_hbm,
          idx_u, idx_i,
          b0, b1, b2, b3, b4, b5, b6, b7,
          b8, b9, b10, b11, b12, b13, b14, b15,
          outpad, trans_u, trans_i, xui_v,
          sem_g, sem_o):
    wid = lax.axis_index("s") * NC + lax.axis_index("c")
    base = wid * BPW
    bufs = [b0, b1, b2, b3, b4, b5, b6, b7,
            b8, b9, b10, b11, b12, b13, b14, b15]

    pltpu.sync_copy(users_hbm.at[pl.ds(base, BPW)], idx_u)
    pltpu.sync_copy(items_hbm.at[pl.ds(base, BPW)], idx_i)

    cvec = lax.iota(jnp.int32, L)

    def do_table(idx_ref, table_t, pad_hbm, trans):
        for w in range(2):
            woff = w * WAVE

            def ring_iter(i, carry):
                s = woff + i * RING
                v = idx_ref[pl.ds(s, RING)]
                copies = []
                lanes = []
                for b in range(RING):
                    r = v[b]
                    boff = pl.multiple_of(
                        lax.shift_left(lax.shift_right_logical(r, 7), 7), 128)
                    lanes.append(lax.bitwise_and(r, jnp.int32(127)))
                    copies.append(pltpu.async_copy(
                        table_t.at[:, pl.ds(boff, 128)], bufs[b], sem_g))
                for b in range(RING):
                    copies[b].wait()
                    j_local = i * RING + b
                    jv = jnp.full((L,), j_local, jnp.int32)
                    vals = plsc.load_gather(
                        bufs[b], [cvec, jnp.full((L,), lanes[b], jnp.int32)])
                    plsc.store_scatter(outpad, [jv, cvec], vals)
                    plsc.store_scatter(
                        trans, [cvec * BPW + (s + b)], vals)
                return carry

            lax.fori_loop(0, WAVE // RING, ring_iter, 0)

            pltpu.sync_copy(
                outpad, pad_hbm.at[pl.ds(base + woff, WAVE), :])

    do_table(idx_u, gu_t, gu_pad_hbm, trans_u)
    do_table(idx_i, gi_t, gi_pad_hbm, trans_i)

    def blk(g, carry):
        s = g * L
        acc = trans_u[pl.ds(s, L)] * trans_i[pl.ds(s, L)]
        for c in range(1, K):
            acc = acc + (trans_u[pl.ds(c * BPW + s, L)]
                         * trans_i[pl.ds(c * BPW + s, L)])
        xui_v[pl.ds(s, L)] = acc
        return carry

    lax.fori_loop(0, BPW // L, blk, 0)

    pltpu.sync_copy(xui_v, xui_hbm.at[pl.ds(base, BPW)])


@jax.jit
def _run(users, items, gu_t, gi_t):
    mesh = plsc.VectorSubcoreMesh(core_axis_name="c", subcore_axis_name="s")
    f = pl.kernel(
        _body,
        mesh=mesh,
        compiler_params=pltpu.CompilerParams(
            needs_layout_passes=False, use_tc_tiling_on_sc=True),
        out_type=(
            jax.ShapeDtypeStruct((BATCH,), jnp.float32),
            jax.ShapeDtypeStruct((BATCH, GW), jnp.float32),
            jax.ShapeDtypeStruct((BATCH, GW), jnp.float32),
        ),
        scratch_types=(
            [pltpu.VMEM((BPW,), jnp.int32),
             pltpu.VMEM((BPW,), jnp.int32)]
            + [pltpu.VMEM((K, 128), jnp.float32) for _ in range(RING)]
            + [pltpu.VMEM((WAVE, GW), jnp.float32),
               pltpu.VMEM((K * BPW,), jnp.float32),
               pltpu.VMEM((K * BPW,), jnp.float32),
               pltpu.VMEM((BPW,), jnp.float32),
               pltpu.SemaphoreType.DMA,
               pltpu.SemaphoreType.DMA]
        ),
    )
    return f(users, items, gu_t, gi_t)


def kernel(inputs, Gu, Gi):
    users = inputs[0]
    items = inputs[1]
    xui, gu_pad, gi_pad = _run(users, items, Gu.T, Gi.T)
    return xui, gu_pad[:, :K], gi_pad[:, :K]


# speedup vs baseline: 16.3118x; 1.1277x over previous
import jax
import jax.numpy as jnp
from jax import lax
from jax.experimental import pallas as pl
from jax.experimental.pallas import tpu as pltpu
from jax.experimental.pallas import tpu_sc as plsc

BATCH = 16384
K = 16
NROWS = 1_000_000
GW = 128

_info = plsc.get_sparse_core_info()
NC, NS, L = _info.num_cores, _info.num_subcores, _info.num_lanes
NW = NC * NS
BPW = BATCH // NW
WAVE = 256
NG = BPW // L


def _body(users_hbm, items_hbm, gu_t, gi_t,
          xui_hbm, gu_pad_hbm, gi_pad_hbm,
          idx_u, idx_i, *rest):
    bufs = rest[:32]
    outpad, trans_u, trans_i, xui_v, sem_a, sem_b = rest[32:]
    half_a = bufs[:16]
    half_b = bufs[16:]

    wid = lax.axis_index("s") * NC + lax.axis_index("c")
    base = wid * BPW

    pltpu.sync_copy(users_hbm.at[pl.ds(base, BPW)], idx_u)
    pltpu.sync_copy(items_hbm.at[pl.ds(base, BPW)], idx_i)

    cvec = lax.iota(jnp.int32, L)

    def issue(idx_ref, table_t, g, half, sem):
        s = jnp.minimum(g, NG - 1) * L
        v = idx_ref[pl.ds(s, L)]
        for b in range(L):
            boff = pl.multiple_of(
                lax.shift_left(
                    lax.shift_right_logical(v[b], 7), 7), 128)
            pltpu.async_copy(
                table_t.at[:, pl.ds(boff, 128)], half[b], sem)

    def drain(sem):
        pltpu.make_async_copy(
            gu_pad_hbm.at[pl.ds(0, WAVE), :], outpad, sem).wait()

    def extract(idx_ref, trans, g, woff, half):
        s = g * L
        v = idx_ref[pl.ds(s, L)]
        for b in range(L):
            lane = lax.bitwise_and(v[b], jnp.int32(127))
            vals = plsc.load_gather(
                half[b], [cvec, jnp.full((L,), lane, jnp.int32)])
            plsc.store_scatter(
                outpad, [jnp.full((L,), s - woff + b, jnp.int32), cvec],
                vals)
            plsc.store_scatter(trans, [cvec * BPW + (s + b)], vals)

    def do_table(idx_ref, table_t, pad_hbm, trans):
        issue(idx_ref, table_t, 0, half_a, sem_a)
        for w in range(2):
            woff = w * WAVE
            kbase = w * 8

            def body(k, carry):
                ga = (kbase + k) * 2
                issue(idx_ref, table_t, ga + 1, half_b, sem_b)
                drain(sem_a)
                extract(idx_ref, trans, ga, woff, half_a)
                issue(idx_ref, table_t, ga + 2, half_a, sem_a)
                drain(sem_b)
                extract(idx_ref, trans, ga + 1, woff, half_b)
                return carry

            lax.fori_loop(0, 8, body, 0)
            pltpu.sync_copy(
                outpad, pad_hbm.at[pl.ds(base + woff, WAVE), :])
        drain(sem_a)

    do_table(idx_u, gu_t, gu_pad_hbm, trans_u)
    do_table(idx_i, gi_t, gi_pad_hbm, trans_i)

    def blk(g, carry):
        s = g * L
        acc = trans_u[pl.ds(s, L)] * trans_i[pl.ds(s, L)]
        for c in range(1, K):
            acc = acc + (trans_u[pl.ds(c * BPW + s, L)]
                         * trans_i[pl.ds(c * BPW + s, L)])
        xui_v[pl.ds(s, L)] = acc
        return carry

    lax.fori_loop(0, NG, blk, 0)

    pltpu.sync_copy(xui_v, xui_hbm.at[pl.ds(base, BPW)])


@jax.jit
def _run(users, items, gu_t, gi_t):
    mesh = plsc.VectorSubcoreMesh(core_axis_name="c", subcore_axis_name="s")
    f = pl.kernel(
        _body,
        mesh=mesh,
        compiler_params=pltpu.CompilerParams(
            needs_layout_passes=False, use_tc_tiling_on_sc=True),
        out_type=(
            jax.ShapeDtypeStruct((BATCH,), jnp.float32),
            jax.ShapeDtypeStruct((BATCH, GW), jnp.float32),
            jax.ShapeDtypeStruct((BATCH, GW), jnp.float32),
        ),
        scratch_types=(
            [pltpu.VMEM((BPW,), jnp.int32),
             pltpu.VMEM((BPW,), jnp.int32)]
            + [pltpu.VMEM((K, 128), jnp.float32) for _ in range(32)]
            + [pltpu.VMEM((WAVE, GW), jnp.float32),
               pltpu.VMEM((K * BPW,), jnp.float32),
               pltpu.VMEM((K * BPW,), jnp.float32),
               pltpu.VMEM((BPW,), jnp.float32),
               pltpu.SemaphoreType.DMA,
               pltpu.SemaphoreType.DMA]
        ),
    )
    return f(users, items, gu_t, gi_t)


def kernel(inputs, Gu, Gi):
    users = inputs[0]
    items = inputs[1]
    xui, gu_pad, gi_pad = _run(users, items, Gu.T, Gi.T)
    return xui, gu_pad[:, :K], gi_pad[:, :K]


# speedup vs baseline: 17.7333x; 1.0871x over previous
import jax
import jax.numpy as jnp
from jax import lax
from jax.experimental import pallas as pl
from jax.experimental.pallas import tpu as pltpu
from jax.experimental.pallas import tpu_sc as plsc

BATCH = 16384
K = 16
NROWS = 1_000_000

_info = plsc.get_sparse_core_info()
NC, NS, L = _info.num_cores, _info.num_subcores, _info.num_lanes
NW = NC * NS
BPW = BATCH // NW
NG = BPW // L


def _body(users_hbm, items_hbm, gu_t, gi_t,
          xui_hbm, gu_out_hbm, gi_out_hbm,
          idx_u, idx_i, *rest):
    bufs = rest[:32]
    trans_u, trans_i, xui_v, sem_a, sem_b = rest[32:]
    half_a = bufs[:16]
    half_b = bufs[16:]

    wid = lax.axis_index("s") * NC + lax.axis_index("c")
    base = wid * BPW

    pltpu.sync_copy(users_hbm.at[pl.ds(base, BPW)], idx_u)
    pltpu.sync_copy(items_hbm.at[pl.ds(base, BPW)], idx_i)

    cvec = lax.iota(jnp.int32, L)

    def issue(idx_ref, table_t, g, half, sem):
        s = jnp.minimum(g, NG - 1) * L
        v = idx_ref[pl.ds(s, L)]
        for b in range(L):
            boff = pl.multiple_of(
                lax.shift_left(
                    lax.shift_right_logical(v[b], 7), 7), 128)
            pltpu.async_copy(
                table_t.at[:, pl.ds(boff, 128)], half[b], sem)

    def drain(half, sem):
        for b in range(L):
            pltpu.make_async_copy(
                gu_t.at[:, pl.ds(0, 128)], half[b], sem).wait()

    def extract(idx_ref, trans, g, half):
        s = g * L
        v = idx_ref[pl.ds(s, L)]
        for b in range(L):
            lane = lax.bitwise_and(v[b], jnp.int32(127))
            vals = plsc.load_gather(
                half[b], [cvec, jnp.full((L,), lane, jnp.int32)])
            plsc.store_scatter(
                trans, [cvec, jnp.full((L,), s + b, jnp.int32)], vals)

    def do_table(idx_ref, table_t, out_hbm, trans):
        issue(idx_ref, table_t, 0, half_a, sem_a)

        def body(k, carry):
            ga = k * 2
            issue(idx_ref, table_t, ga + 1, half_b, sem_b)
            drain(half_a, sem_a)
            extract(idx_ref, trans, ga, half_a)
            issue(idx_ref, table_t, ga + 2, half_a, sem_a)
            drain(half_b, sem_b)
            extract(idx_ref, trans, ga + 1, half_b)
            return carry

        lax.fori_loop(0, NG // 2, body, 0)
        drain(half_a, sem_a)
        pltpu.sync_copy(trans, out_hbm.at[:, pl.ds(base, BPW)])

    do_table(idx_u, gu_t, gu_out_hbm, trans_u)
    do_table(idx_i, gi_t, gi_out_hbm, trans_i)

    def blk(g, carry):
        s = g * L
        jvec = s + cvec
        acc = (plsc.load_gather(trans_u, [jnp.zeros((L,), jnp.int32), jvec])
               * plsc.load_gather(trans_i, [jnp.zeros((L,), jnp.int32), jvec]))
        for c in range(1, K):
            col = jnp.full((L,), c, jnp.int32)
            acc = acc + (plsc.load_gather(trans_u, [col, jvec])
                         * plsc.load_gather(trans_i, [col, jvec]))
        xui_v[pl.ds(s, L)] = acc
        return carry

    lax.fori_loop(0, NG, blk, 0)

    pltpu.sync_copy(xui_v, xui_hbm.at[pl.ds(base, BPW)])


@jax.jit
def _run(users, items, gu_t, gi_t):
    mesh = plsc.VectorSubcoreMesh(core_axis_name="c", subcore_axis_name="s")
    f = pl.kernel(
        _body,
        mesh=mesh,
        compiler_params=pltpu.CompilerParams(
            needs_layout_passes=False, use_tc_tiling_on_sc=True),
        out_type=(
            jax.ShapeDtypeStruct((BATCH,), jnp.float32),
            jax.ShapeDtypeStruct((K, BATCH), jnp.float32),
            jax.ShapeDtypeStruct((K, BATCH), jnp.float32),
        ),
        scratch_types=(
            [pltpu.VMEM((BPW,), jnp.int32),
             pltpu.VMEM((BPW,), jnp.int32)]
            + [pltpu.VMEM((K, 128), jnp.float32) for _ in range(32)]
            + [pltpu.VMEM((K, BPW), jnp.float32),
               pltpu.VMEM((K, BPW), jnp.float32),
               pltpu.VMEM((BPW,), jnp.float32),
               pltpu.SemaphoreType.DMA,
               pltpu.SemaphoreType.DMA]
        ),
    )
    return f(users, items, gu_t, gi_t)


def kernel(inputs, Gu, Gi):
    users = inputs[0]
    items = inputs[1]
    xui, gu_out_t, gi_out_t = _run(users, items, Gu.T, Gi.T)
    return xui, gu_out_t.T, gi_out_t.T
